# Initial kernel scaffold; baseline (speedup 1.0000x reference)
import jax
import jax.numpy as jnp
from jax.experimental import pallas as pl


def kernel(n_verts, time, mass, evals, evecs, gradX_vals, gradY_vals, W_in, b_in, diff_time, gradA_real, gradA_imag, W0, b0, W1, b1, W_out, b_out, grad_rows, grad_cols, faces):
    raise NotImplementedError("write your pallas kernel here")


# speedup vs baseline: 3.0065x; 3.0065x over previous
import jax
import jax.numpy as jnp
from jax.experimental import pallas as pl

_BN = 1024
_K = 128
_C = 128


def _x0_body(v_ref, w_ref, br_ref, m_ref, x_ref, mx_ref):
    x = jnp.dot(v_ref[...], w_ref[...], preferred_element_type=jnp.float32)
    x = x + br_ref[...]
    x_ref[...] = x
    mx_ref[...] = m_ref[...] * x


def _spec_body(ev_ref, mx_ref, o_ref):
    @pl.when(pl.program_id(0) == 0)
    def _():
        o_ref[...] = jnp.zeros_like(o_ref)

    o_ref[...] += jax.lax.dot_general(
        ev_ref[...], mx_ref[...], (((0,), (0,)), ((), ())),
        preferred_element_type=jnp.float32)


def _block_body(x_ref, ev_ref, gxm_ref, gym_ref, m_ref, xs_ref, evc_ref,
                dt_ref, ar_ref, ai_ref, w0a_ref, w0b_ref, w0c_ref, b0_ref,
                w1_ref, b1_ref, xn_ref, mx_ref):
    T = jnp.exp(-evc_ref[...] * dt_ref[...]) * xs_ref[...]
    xd = jnp.dot(ev_ref[...], T, preferred_element_type=jnp.float32)
    gx = jnp.dot(gxm_ref[...], T, preferred_element_type=jnp.float32)
    gy = jnp.dot(gym_ref[...], T, preferred_element_type=jnp.float32)
    ar = ar_ref[...]
    ai = ai_ref[...]
    br = (jnp.dot(gx, ar, preferred_element_type=jnp.float32)
          - jnp.dot(gy, ai, preferred_element_type=jnp.float32))
    bi = (jnp.dot(gx, ai, preferred_element_type=jnp.float32)
          + jnp.dot(gy, ar, preferred_element_type=jnp.float32))
    xg = jnp.tanh(gx * br + gy * bi)
    x = x_ref[...]
    h = (jnp.dot(x, w0a_ref[...], preferred_element_type=jnp.float32)
         + jnp.dot(xd, w0b_ref[...], preferred_element_type=jnp.float32)
         + jnp.dot(xg, w0c_ref[...], preferred_element_type=jnp.float32)
         + b0_ref[...])
    h = jnp.maximum(h, 0.0)
    h = jnp.dot(h, w1_ref[...], preferred_element_type=jnp.float32) + b1_ref[...]
    xn = x + h
    xn_ref[...] = xn
    mx_ref[...] = m_ref[...] * xn


def _out_body(x_ref, wo_ref, bo_ref, o_ref):
    o_ref[...] = (jnp.dot(x_ref[...], wo_ref[...],
                          preferred_element_type=jnp.float32) + bo_ref[...])


def kernel(n_verts, time, mass, evals, evecs, gradX_vals, gradY_vals, W_in,
           b_in, diff_time, gradA_real, gradA_imag, W0, b0, W1, b1, W_out,
           b_out, grad_rows, grad_cols, faces):
    n = n_verts.shape[0]
    npad = ((n + _BN - 1) // _BN) * _BN
    grid = npad // _BN

    src = evecs[grad_cols]
    GX = jnp.zeros((n, _K), jnp.float32).at[grad_rows].add(
        gradX_vals[:, None] * src)
    GY = jnp.zeros((n, _K), jnp.float32).at[grad_rows].add(
        gradY_vals[:, None] * src)

    pr = npad - n
    evecs_p = jnp.pad(evecs, ((0, pr), (0, 0)))
    GX = jnp.pad(GX, ((0, pr), (0, 0)))
    GY = jnp.pad(GY, ((0, pr), (0, 0)))
    mass2d = jnp.pad(mass, (0, pr))[:, None]

    freqs = 2.0 ** jnp.arange(10, dtype=jnp.float32)
    ang = time * freqs
    sc = jnp.stack([jnp.sin(ang), jnp.cos(ang)], axis=1).reshape(-1)
    temb = jnp.concatenate([time, sc])
    brow = (temb[None, :] @ W_in[3:] + b_in[None, :])
    verts_p = jnp.pad(n_verts, ((0, pr), (0, 125)))
    W3 = jnp.pad(W_in[:3], ((0, 125), (0, 0)))

    row = pl.BlockSpec((_BN, _C), lambda i: (i, 0))
    mrow = pl.BlockSpec((_BN, 1), lambda i: (i, 0))
    full = pl.BlockSpec((_K, _C), lambda i: (0, 0))
    col1 = pl.BlockSpec((_K, 1), lambda i: (0, 0))
    row1 = pl.BlockSpec((1, _C), lambda i: (0, 0))

    rowt = jax.ShapeDtypeStruct((npad, _C), jnp.float32)
    spect = jax.ShapeDtypeStruct((_K, _C), jnp.float32)

    x, mx = pl.pallas_call(
        _x0_body, grid=(grid,),
        in_specs=[row, full, row1, mrow],
        out_specs=(row, row),
        out_shape=(rowt, rowt),
    )(verts_p, W3, brow, mass2d)

    evals_col = evals[:, None]
    dts = jnp.clip(diff_time, 1e-8, None)

    for blk in range(4):
        xs = pl.pallas_call(
            _spec_body, grid=(grid,),
            in_specs=[row, row],
            out_specs=full,
            out_shape=spect,
        )(evecs_p, mx)
        x, mx = pl.pallas_call(
            _block_body, grid=(grid,),
            in_specs=[row, row, row, row, mrow, full, col1, row1, full, full,
                      full, full, full, row1, full, row1],
            out_specs=(row, row),
            out_shape=(rowt, rowt),
        )(x, evecs_p, GX, GY, mass2d, xs, evals_col, dts[blk][None, :],
          gradA_real[blk], gradA_imag[blk],
          W0[blk][:_C], W0[blk][_C:2 * _C], W0[blk][2 * _C:], b0[blk][None, :],
          W1[blk], b1[blk][None, :])

    Wo = jnp.pad(W_out, ((0, 0), (0, _C - 9)))
    bo = jnp.pad(b_out, (0, _C - 9))[None, :]
    xo = pl.pallas_call(
        _out_body, grid=(grid,),
        in_specs=[row, full, row1],
        out_specs=row,
        out_shape=rowt,
    )(x, Wo, bo)

    xo = xo[:n, :9]
    out = (xo[faces[:, 0]] + xo[faces[:, 1]] + xo[faces[:, 2]]) * (1.0 / 3.0)
    return out


# speedup vs baseline: 3.2474x; 1.0801x over previous
import jax
import jax.numpy as jnp
from jax.experimental import pallas as pl

_BN = 1024
_K = 128
_C = 128


def _x0_body(v_ref, w_ref, br_ref, m_ref, x_ref, mx_ref):
    x = jnp.dot(v_ref[...], w_ref[...], preferred_element_type=jnp.float32)
    x = x + br_ref[...]
    x_ref[...] = x
    mx_ref[...] = m_ref[...] * x


def _spec_body(ev_ref, mx_ref, o_ref):
    @pl.when(pl.program_id(0) == 0)
    def _():
        o_ref[...] = jnp.zeros_like(o_ref)

    o_ref[...] += jax.lax.dot_general(
        ev_ref[...], mx_ref[...], (((0,), (0,)), ((), ())),
        preferred_element_type=jnp.float32)


def _block_body(x_ref, ev_ref, gxy_ref, m_ref, xs_ref, evc_ref,
                dt_ref, ar_ref, ai_ref, w0a_ref, w0b_ref, w0c_ref, b0_ref,
                w1_ref, b1_ref, xn_ref, mx_ref):
    T = jnp.exp(-evc_ref[...] * dt_ref[...]) * xs_ref[...]
    xd = jnp.dot(ev_ref[...], T, preferred_element_type=jnp.float32)
    gx = jnp.dot(gxy_ref[:, :_K], T, preferred_element_type=jnp.float32)
    gy = jnp.dot(gxy_ref[:, _K:], T, preferred_element_type=jnp.float32)
    ar = ar_ref[...]
    ai = ai_ref[...]
    br = (jnp.dot(gx, ar, preferred_element_type=jnp.float32)
          - jnp.dot(gy, ai, preferred_element_type=jnp.float32))
    bi = (jnp.dot(gx, ai, preferred_element_type=jnp.float32)
          + jnp.dot(gy, ar, preferred_element_type=jnp.float32))
    xg = jnp.tanh(gx * br + gy * bi)
    x = x_ref[...]
    h = (jnp.dot(x, w0a_ref[...], preferred_element_type=jnp.float32)
         + jnp.dot(xd, w0b_ref[...], preferred_element_type=jnp.float32)
         + jnp.dot(xg, w0c_ref[...], preferred_element_type=jnp.float32)
         + b0_ref[...])
    h = jnp.maximum(h, 0.0)
    h = jnp.dot(h, w1_ref[...], preferred_element_type=jnp.float32) + b1_ref[...]
    xn = x + h
    xn_ref[...] = xn
    mx_ref[...] = m_ref[...] * xn


def _out_body(x_ref, wo_ref, bo_ref, o_ref):
    o_ref[...] = (jnp.dot(x_ref[...], wo_ref[...],
                          preferred_element_type=jnp.float32) + bo_ref[...])


def kernel(n_verts, time, mass, evals, evecs, gradX_vals, gradY_vals, W_in,
           b_in, diff_time, gradA_real, gradA_imag, W0, b0, W1, b1, W_out,
           b_out, grad_rows, grad_cols, faces):
    n = n_verts.shape[0]
    npad = ((n + _BN - 1) // _BN) * _BN
    grid = npad // _BN

    src = evecs[grad_cols]
    contrib = jnp.concatenate(
        [gradX_vals[:, None] * src, gradY_vals[:, None] * src], axis=1)
    GXY = jnp.zeros((npad, 2 * _K), jnp.float32).at[grad_rows].add(contrib)

    pr = npad - n
    evecs_p = jnp.pad(evecs, ((0, pr), (0, 0)))
    mass2d = jnp.pad(mass, (0, pr))[:, None]

    freqs = 2.0 ** jnp.arange(10, dtype=jnp.float32)
    ang = time * freqs
    sc = jnp.stack([jnp.sin(ang), jnp.cos(ang)], axis=1).reshape(-1)
    temb = jnp.concatenate([time, sc])
    brow = (temb[None, :] @ W_in[3:] + b_in[None, :])
    verts_p = jnp.pad(n_verts, ((0, pr), (0, 125)))
    W3 = jnp.pad(W_in[:3], ((0, 125), (0, 0)))

    row = pl.BlockSpec((_BN, _C), lambda i: (i, 0))
    mrow = pl.BlockSpec((_BN, 1), lambda i: (i, 0))
    full = pl.BlockSpec((_K, _C), lambda i: (0, 0))
    col1 = pl.BlockSpec((_K, 1), lambda i: (0, 0))
    row1 = pl.BlockSpec((1, _C), lambda i: (0, 0))

    rowt = jax.ShapeDtypeStruct((npad, _C), jnp.float32)
    spect = jax.ShapeDtypeStruct((_K, _C), jnp.float32)

    x, mx = pl.pallas_call(
        _x0_body, grid=(grid,),
        in_specs=[row, full, row1, mrow],
        out_specs=(row, row),
        out_shape=(rowt, rowt),
    )(verts_p, W3, brow, mass2d)

    evals_col = evals[:, None]
    dts = jnp.clip(diff_time, 1e-8, None)

    for blk in range(4):
        xs = pl.pallas_call(
            _spec_body, grid=(grid,),
            in_specs=[row, row],
            out_specs=full,
            out_shape=spect,
        )(evecs_p, mx)
        x, mx = pl.pallas_call(
            _block_body, grid=(grid,),
            in_specs=[row, row, pl.BlockSpec((_BN, 2 * _K), lambda i: (i, 0)),
                      mrow, full, col1, row1, full, full,
                      full, full, full, row1, full, row1],
            out_specs=(row, row),
            out_shape=(rowt, rowt),
        )(x, evecs_p, GXY, mass2d, xs, evals_col, dts[blk][None, :],
          gradA_real[blk], gradA_imag[blk],
          W0[blk][:_C], W0[blk][_C:2 * _C], W0[blk][2 * _C:], b0[blk][None, :],
          W1[blk], b1[blk][None, :])

    Wo = jnp.pad(W_out, ((0, 0), (0, _C - 9)))
    bo = jnp.pad(b_out, (0, _C - 9))[None, :]
    xo = pl.pallas_call(
        _out_body, grid=(grid,),
        in_specs=[row, full, row1],
        out_specs=row,
        out_shape=rowt,
    )(x, Wo, bo)

    xo = xo[:n, :9]
    out = (xo[faces[:, 0]] + xo[faces[:, 1]] + xo[faces[:, 2]]) * (1.0 / 3.0)
    return out


# speedup vs baseline: 3.3078x; 1.0186x over previous
import jax
import jax.numpy as jnp
from jax.experimental import pallas as pl

_BN = 2048
_K = 128
_C = 128


def _x0_body(v_ref, w_ref, br_ref, m_ref, x_ref, mx_ref):
    x = jnp.dot(v_ref[...], w_ref[...], preferred_element_type=jnp.float32)
    x = x + br_ref[...]
    x_ref[...] = x
    mx_ref[...] = m_ref[...] * x


def _spec_body(ev_ref, mx_ref, o_ref):
    @pl.when(pl.program_id(0) == 0)
    def _():
        o_ref[...] = jnp.zeros_like(o_ref)

    o_ref[...] += jax.lax.dot_general(
        ev_ref[...], mx_ref[...], (((0,), (0,)), ((), ())),
        preferred_element_type=jnp.float32)


def _block_body(x_ref, ev_ref, gxy_ref, m_ref, xs_ref, evc_ref,
                dt_ref, ar_ref, ai_ref, w0a_ref, w0b_ref, w0c_ref, b0_ref,
                w1_ref, b1_ref, xn_ref, mx_ref):
    T = jnp.exp(-evc_ref[...] * dt_ref[...]) * xs_ref[...]
    xd = jnp.dot(ev_ref[...], T, preferred_element_type=jnp.float32)
    gx = jnp.dot(gxy_ref[:, :_K], T, preferred_element_type=jnp.float32)
    gy = jnp.dot(gxy_ref[:, _K:], T, preferred_element_type=jnp.float32)
    ar = ar_ref[...]
    ai = ai_ref[...]
    br = (jnp.dot(gx, ar, preferred_element_type=jnp.float32)
          - jnp.dot(gy, ai, preferred_element_type=jnp.float32))
    bi = (jnp.dot(gx, ai, preferred_element_type=jnp.float32)
          + jnp.dot(gy, ar, preferred_element_type=jnp.float32))
    xg = jnp.tanh(gx * br + gy * bi)
    x = x_ref[...]
    h = (jnp.dot(x, w0a_ref[...], preferred_element_type=jnp.float32)
         + jnp.dot(xd, w0b_ref[...], preferred_element_type=jnp.float32)
         + jnp.dot(xg, w0c_ref[...], preferred_element_type=jnp.float32)
         + b0_ref[...])
    h = jnp.maximum(h, 0.0)
    h = jnp.dot(h, w1_ref[...], preferred_element_type=jnp.float32) + b1_ref[...]
    xn = x + h
    xn_ref[...] = xn
    mx_ref[...] = m_ref[...] * xn


def _out_body(x_ref, wo_ref, bo_ref, o_ref):
    o_ref[...] = (jnp.dot(x_ref[...], wo_ref[...],
                          preferred_element_type=jnp.float32) + bo_ref[...])


def kernel(n_verts, time, mass, evals, evecs, gradX_vals, gradY_vals, W_in,
           b_in, diff_time, gradA_real, gradA_imag, W0, b0, W1, b1, W_out,
           b_out, grad_rows, grad_cols, faces):
    n = n_verts.shape[0]
    npad = ((n + _BN - 1) // _BN) * _BN
    grid = npad // _BN

    src = evecs[grad_cols]
    contrib = jnp.concatenate(
        [gradX_vals[:, None] * src, gradY_vals[:, None] * src], axis=1)
    GXY = jnp.zeros((npad, 2 * _K), jnp.float32).at[grad_rows].add(contrib)

    pr = npad - n
    evecs_p = jnp.pad(evecs, ((0, pr), (0, 0)))
    mass2d = jnp.pad(mass, (0, pr))[:, None]

    freqs = 2.0 ** jnp.arange(10, dtype=jnp.float32)
    ang = time * freqs
    sc = jnp.stack([jnp.sin(ang), jnp.cos(ang)], axis=1).reshape(-1)
    temb = jnp.concatenate([time, sc])
    brow = (temb[None, :] @ W_in[3:] + b_in[None, :])
    verts_p = jnp.pad(n_verts, ((0, pr), (0, 125)))
    W3 = jnp.pad(W_in[:3], ((0, 125), (0, 0)))

    row = pl.BlockSpec((_BN, _C), lambda i: (i, 0))
    mrow = pl.BlockSpec((_BN, 1), lambda i: (i, 0))
    full = pl.BlockSpec((_K, _C), lambda i: (0, 0))
    col1 = pl.BlockSpec((_K, 1), lambda i: (0, 0))
    row1 = pl.BlockSpec((1, _C), lambda i: (0, 0))

    rowt = jax.ShapeDtypeStruct((npad, _C), jnp.float32)
    spect = jax.ShapeDtypeStruct((_K, _C), jnp.float32)

    x, mx = pl.pallas_call(
        _x0_body, grid=(grid,),
        in_specs=[row, full, row1, mrow],
        out_specs=(row, row),
        out_shape=(rowt, rowt),
    )(verts_p, W3, brow, mass2d)

    evals_col = evals[:, None]
    dts = jnp.clip(diff_time, 1e-8, None)

    for blk in range(4):
        xs = pl.pallas_call(
            _spec_body, grid=(grid,),
            in_specs=[row, row],
            out_specs=full,
            out_shape=spect,
        )(evecs_p, mx)
        x, mx = pl.pallas_call(
            _block_body, grid=(grid,),
            in_specs=[row, row, pl.BlockSpec((_BN, 2 * _K), lambda i: (i, 0)),
                      mrow, full, col1, row1, full, full,
                      full, full, full, row1, full, row1],
            out_specs=(row, row),
            out_shape=(rowt, rowt),
        )(x, evecs_p, GXY, mass2d, xs, evals_col, dts[blk][None, :],
          gradA_real[blk], gradA_imag[blk],
          W0[blk][:_C], W0[blk][_C:2 * _C], W0[blk][2 * _C:], b0[blk][None, :],
          W1[blk], b1[blk][None, :])

    Wo = jnp.pad(W_out, ((0, 0), (0, _C - 9)))
    bo = jnp.pad(b_out, (0, _C - 9))[None, :]
    xo = pl.pallas_call(
        _out_body, grid=(grid,),
        in_specs=[row, full, row1],
        out_specs=row,
        out_shape=rowt,
    )(x, Wo, bo)

    xo = xo[:n, :9]
    out = (xo[faces[:, 0]] + xo[faces[:, 1]] + xo[faces[:, 2]]) * (1.0 / 3.0)
    return out
